# baseline (device time: 21876 ns/iter reference)
import jax
import jax.numpy as jnp
from jax import lax
from jax.experimental import pallas as pl
from jax.experimental.pallas import tpu as pltpu


def kernel(Q, K, V):
    B, SQ, H, D = Q.shape
    SKV = K.shape[1]
    HD = H * D
    scale = D ** -0.5

    K2 = K.reshape(B, SKV, HD)
    V2 = V.reshape(B, SKV, HD)
    Q2 = Q.reshape(B * H, D)

    def body(q_ref, k_ref, v_ref, o_ref, send_buf, recv_buf,
             kbuf, vbuf, ksem, vsem, send_sem, recv_sem):
        ix = lax.axis_index("x")
        iy = lax.axis_index("y")
        iz = lax.axis_index("z")
        nbr = (1 - ix, iy, iz)

        barrier = pltpu.get_barrier_semaphore()
        pl.semaphore_signal(
            barrier, inc=1, device_id=nbr, device_id_type=pl.DeviceIdType.MESH
        )

        def start_copy(b, slot):
            pltpu.make_async_copy(k_ref.at[b], kbuf.at[slot], ksem.at[slot]).start()
            pltpu.make_async_copy(v_ref.at[b], vbuf.at[slot], vsem.at[slot]).start()

        def wait_copy(b, slot):
            pltpu.make_async_copy(k_ref.at[b], kbuf.at[slot], ksem.at[slot]).wait()
            pltpu.make_async_copy(v_ref.at[b], vbuf.at[slot], vsem.at[slot]).wait()

        start_copy(0, 0)

        q2 = q_ref[...]
        qrep = jnp.concatenate([q2] * H, axis=1)
        row_h = lax.broadcasted_iota(jnp.int32, (B * H, HD), 0) % H
        col_b = lax.broadcasted_iota(jnp.int32, (B * H, HD), 1) // D
        qblk = jnp.where(row_h == col_b, qrep, 0.0).astype(jnp.bfloat16)

        for b in range(B):
            slot = b % 2
            if b + 1 < B:
                start_copy(b + 1, (b + 1) % 2)
            wait_copy(b, slot)
            kb = kbuf[slot].astype(jnp.bfloat16)
            vb = vbuf[slot].astype(jnp.bfloat16)
            s = lax.dot_general(
                qblk[b * H:(b + 1) * H, :], kb, (((1,), (1,)), ((), ())),
                preferred_element_type=jnp.float32,
            ) * scale
            p = jnp.exp(s)
            l = jnp.sum(p, axis=1, keepdims=True)
            r = lax.dot_general(
                p.astype(jnp.bfloat16), vb, (((1,), (0,)), ((), ())),
                preferred_element_type=jnp.float32,
            )
            for h in range(H):
                send_buf[pl.ds(b * H + h, 1), pl.ds(0, D)] = (
                    r[h:h + 1, h * D:(h + 1) * D]
                )
            send_buf[pl.ds(b * H, H), pl.ds(D, D)] = jnp.broadcast_to(l, (H, D))

        pl.semaphore_wait(barrier, 1)
        rdma = pltpu.make_async_remote_copy(
            src_ref=send_buf,
            dst_ref=recv_buf,
            send_sem=send_sem,
            recv_sem=recv_sem,
            device_id=nbr,
            device_id_type=pl.DeviceIdType.MESH,
        )
        rdma.start()
        rdma.wait()

        tot = send_buf[...] + recv_buf[...]
        o_all = tot[:, :D] / tot[:, D:D + 1]
        o_ref[...] = o_all.reshape(B, SQ, H, D)

    return pl.pallas_call(
        body,
        out_shape=jax.ShapeDtypeStruct((B, SQ, H, D), jnp.float32),
        in_specs=[
            pl.BlockSpec(memory_space=pltpu.VMEM),
            pl.BlockSpec(memory_space=pltpu.MemorySpace.HBM),
            pl.BlockSpec(memory_space=pltpu.MemorySpace.HBM),
        ],
        out_specs=pl.BlockSpec(memory_space=pltpu.VMEM),
        scratch_shapes=[
            pltpu.VMEM((B * H, 2 * D), jnp.float32),
            pltpu.VMEM((B * H, 2 * D), jnp.float32),
            pltpu.VMEM((2, SKV, HD), jnp.float32),
            pltpu.VMEM((2, SKV, HD), jnp.float32),
            pltpu.SemaphoreType.DMA((2,)),
            pltpu.SemaphoreType.DMA((2,)),
            pltpu.SemaphoreType.DMA,
            pltpu.SemaphoreType.DMA,
        ],
        compiler_params=pltpu.CompilerParams(collective_id=0),
    )(Q2, K2, V2)


# device time: 19337 ns/iter; 1.1313x vs baseline; 1.1313x over previous
import jax
import jax.numpy as jnp
from jax import lax
from jax.experimental import pallas as pl
from jax.experimental.pallas import tpu as pltpu


def kernel(Q, K, V):
    B, SQ, H, D = Q.shape
    SKV = K.shape[1]
    HD = H * D
    scale = D ** -0.5

    K2 = K.reshape(B, SKV, HD)
    V2 = V.reshape(B, SKV, HD)
    Q2 = Q.reshape(B * H, D)

    def body(q_ref, k_ref, v_ref, o_ref, send_buf, recv_buf, send_sem, recv_sem):
        ix = lax.axis_index("x")
        iy = lax.axis_index("y")
        iz = lax.axis_index("z")
        nbr = (1 - ix, iy, iz)

        barrier = pltpu.get_barrier_semaphore()
        pl.semaphore_signal(
            barrier, inc=1, device_id=nbr, device_id_type=pl.DeviceIdType.MESH
        )

        q2 = q_ref[...]
        qrep = jnp.concatenate([q2] * H, axis=1)
        row_h = lax.broadcasted_iota(jnp.int32, (B * H, HD), 0) % H
        col_b = lax.broadcasted_iota(jnp.int32, (B * H, HD), 1) // D
        qblk = jnp.where(row_h == col_b, qrep, 0.0)

        for b in range(B):
            kb = k_ref[b]
            vb = v_ref[b]
            s = lax.dot_general(
                qblk[b * H:(b + 1) * H, :], kb, (((1,), (1,)), ((), ())),
                preferred_element_type=jnp.float32,
            ) * scale
            p = jnp.exp(s)
            l = jnp.sum(p, axis=1, keepdims=True)
            r = lax.dot_general(
                p, vb, (((1,), (0,)), ((), ())),
                preferred_element_type=jnp.float32,
            )
            for h in range(H):
                send_buf[pl.ds(b * H + h, 1), pl.ds(0, D)] = (
                    r[h:h + 1, h * D:(h + 1) * D]
                )
            send_buf[pl.ds(b * H, H), pl.ds(D, D)] = jnp.broadcast_to(l, (H, D))

        pl.semaphore_wait(barrier, 1)
        rdma = pltpu.make_async_remote_copy(
            src_ref=send_buf,
            dst_ref=recv_buf,
            send_sem=send_sem,
            recv_sem=recv_sem,
            device_id=nbr,
            device_id_type=pl.DeviceIdType.MESH,
        )
        rdma.start()
        rdma.wait()

        tot = send_buf[...] + recv_buf[...]
        o_all = tot[:, :D] / tot[:, D:D + 1]
        o_ref[...] = o_all.reshape(B, SQ, H, D)

    return pl.pallas_call(
        body,
        out_shape=jax.ShapeDtypeStruct((B, SQ, H, D), jnp.float32),
        in_specs=[pl.BlockSpec(memory_space=pltpu.VMEM)] * 3,
        out_specs=pl.BlockSpec(memory_space=pltpu.VMEM),
        scratch_shapes=[
            pltpu.VMEM((B * H, 2 * D), jnp.float32),
            pltpu.VMEM((B * H, 2 * D), jnp.float32),
            pltpu.SemaphoreType.DMA,
            pltpu.SemaphoreType.DMA,
        ],
        compiler_params=pltpu.CompilerParams(collective_id=0),
    )(Q2, K2, V2)
